# baseline (speedup 1.0000x reference)
import jax
import jax.numpy as jnp
from jax import lax
from jax.experimental import pallas as pl
from jax.experimental.pallas import tpu as pltpu
from jax.experimental.pallas import tpu_sc as plsc

NUM_FIELDS = 26
CARD = 100000
EMBED = 50
EPAD = 56

B, T = 1024, 20
N_ROWS = B * T * NUM_FIELDS

NC, NS = 2, 16
NW = NC * NS
CHUNK = 128
ROWS_PER_W = N_ROWS // NW
CHUNKS_PER_W = ROWS_PER_W // CHUNK


def _gather_body(w_hbm, g_hbm, out_hbm, idx_v, rows_v, sem):
    wid = lax.axis_index("s") * NC + lax.axis_index("c")
    w_base = wid * ROWS_PER_W

    def body(c, carry):
        base = w_base + c * CHUNK
        pltpu.sync_copy(g_hbm.at[pl.ds(base, CHUNK)], idx_v)
        pltpu.async_copy(w_hbm.at[idx_v], rows_v, sem).wait()
        pltpu.sync_copy(rows_v, out_hbm.at[pl.ds(base, CHUNK)])
        return carry

    lax.fori_loop(0, CHUNKS_PER_W, body, 0)


@jax.jit
def _gather(w_pad, g):
    mesh = plsc.VectorSubcoreMesh(core_axis_name="c", subcore_axis_name="s")
    return pl.kernel(
        _gather_body,
        out_type=jax.ShapeDtypeStruct((N_ROWS, EPAD), jnp.float32),
        mesh=mesh,
        scratch_types=[
            pltpu.VMEM((CHUNK,), jnp.int32),
            pltpu.VMEM((CHUNK, EPAD), jnp.float32),
            pltpu.SemaphoreType.DMA,
        ],
        compiler_params=pltpu.CompilerParams(use_tc_tiling_on_sc=False),
    )(w_pad, g)


def kernel(x, W):
    idx = x.reshape(B * T, NUM_FIELDS).astype(jnp.int32)
    g = (idx + jnp.arange(NUM_FIELDS, dtype=jnp.int32) * CARD).reshape(N_ROWS)
    w_flat = W.reshape(NUM_FIELDS * CARD, EMBED)
    w_pad = jnp.concatenate(
        [w_flat, jnp.zeros((NUM_FIELDS * CARD, EPAD - EMBED), jnp.float32)],
        axis=1,
    )
    out = _gather(w_pad, g)
    return out[:, :EMBED].reshape(B, T, NUM_FIELDS * EMBED, 1)


# speedup vs baseline: 1.6269x; 1.6269x over previous
import jax
import jax.numpy as jnp
from jax import lax
from jax.experimental import pallas as pl
from jax.experimental.pallas import tpu as pltpu
from jax.experimental.pallas import tpu_sc as plsc

NUM_FIELDS = 26
CARD = 100000
EMBED = 50
EPAD = 128

B, T = 1024, 20
N_ROWS = B * T * NUM_FIELDS

NC, NS = 2, 16
NW = NC * NS
CHUNK = 128
ROWS_PER_W = N_ROWS // NW
CHUNKS_PER_W = ROWS_PER_W // CHUNK


def _gather_body(w_hbm, g_hbm, out_hbm, idx_v, rows_v, sem):
    wid = lax.axis_index("s") * NC + lax.axis_index("c")
    w_base = wid * ROWS_PER_W

    def body(c, carry):
        base = w_base + c * CHUNK
        pltpu.sync_copy(g_hbm.at[pl.ds(base, CHUNK)], idx_v)
        pltpu.async_copy(w_hbm.at[idx_v], rows_v, sem).wait()
        pltpu.sync_copy(rows_v, out_hbm.at[pl.ds(base, CHUNK)])
        return carry

    lax.fori_loop(0, CHUNKS_PER_W, body, 0)


@jax.jit
def _gather(w_pad, g):
    mesh = plsc.VectorSubcoreMesh(core_axis_name="c", subcore_axis_name="s")
    return pl.kernel(
        _gather_body,
        out_type=jax.ShapeDtypeStruct((N_ROWS, EPAD), jnp.float32),
        mesh=mesh,
        scratch_types=[
            pltpu.VMEM((CHUNK,), jnp.int32),
            pltpu.VMEM((CHUNK, EPAD), jnp.float32),
            pltpu.SemaphoreType.DMA,
        ],
    )(w_pad, g)


def kernel(x, W):
    idx = x.reshape(B * T, NUM_FIELDS).astype(jnp.int32)
    g = (idx + jnp.arange(NUM_FIELDS, dtype=jnp.int32) * CARD).reshape(N_ROWS)
    w_pad = jnp.pad(
        W.reshape(NUM_FIELDS * CARD, EMBED), ((0, 0), (0, EPAD - EMBED))
    )
    out = _gather(w_pad, g)
    return out[:, :EMBED].reshape(B, T, NUM_FIELDS * EMBED, 1)


# speedup vs baseline: 1.6273x; 1.0002x over previous
import jax
import jax.numpy as jnp
from jax import lax
from jax.experimental import pallas as pl
from jax.experimental.pallas import tpu as pltpu
from jax.experimental.pallas import tpu_sc as plsc

NUM_FIELDS = 26
CARD = 100000
EMBED = 50
EPAD = 128

B, T = 1024, 20
N_ROWS = B * T * NUM_FIELDS

NC, NS = 2, 16
NW = NC * NS
CHUNK = 128
ROWS_PER_W = N_ROWS // NW
CHUNKS_PER_W = ROWS_PER_W // CHUNK


def _gather_body(w_hbm, g_hbm, out_hbm, idx_v, rows_v, sem):
    wid = lax.axis_index("s") * NC + lax.axis_index("c")
    w_base = wid * ROWS_PER_W

    def body(c, carry):
        base = w_base + c * CHUNK
        pltpu.sync_copy(g_hbm.at[pl.ds(base, CHUNK)], idx_v)
        pltpu.async_copy(w_hbm.at[idx_v], rows_v, sem).wait()
        pltpu.sync_copy(rows_v, out_hbm.at[pl.ds(base, CHUNK)])
        return carry

    lax.fori_loop(0, CHUNKS_PER_W, body, 0)


@jax.jit
def _gather(w_pad, g):
    mesh = plsc.VectorSubcoreMesh(core_axis_name="c", subcore_axis_name="s")
    return pl.kernel(
        _gather_body,
        out_type=jax.ShapeDtypeStruct((N_ROWS, EPAD), jnp.float32),
        mesh=mesh,
        scratch_types=[
            pltpu.VMEM((CHUNK,), jnp.int32),
            pltpu.VMEM((CHUNK, EPAD), jnp.float32),
            pltpu.SemaphoreType.DMA,
        ],
    )(w_pad, g)


def kernel(x, W):
    idx = x.reshape(B * T, NUM_FIELDS).astype(jnp.int32)
    g = (idx + jnp.arange(NUM_FIELDS, dtype=jnp.int32) * CARD).reshape(N_ROWS)
    w_pad = jnp.pad(W, ((0, 0), (0, 0), (0, EPAD - EMBED))).reshape(
        NUM_FIELDS * CARD, EPAD
    )
    out = _gather(w_pad, g)
    return out[:, :EMBED].reshape(B, T, NUM_FIELDS * EMBED, 1)


# speedup vs baseline: 3.5664x; 2.1917x over previous
import jax
import jax.numpy as jnp
from jax import lax
from jax.experimental import pallas as pl
from jax.experimental.pallas import tpu as pltpu
from jax.experimental.pallas import tpu_sc as plsc

NUM_FIELDS = 26
CARD = 100000
EMBED = 50

B, T = 1024, 20
NBT = B * T
PAIRS = NUM_FIELDS * EMBED

NC, NS = 2, 16
NW = NC * NS
BASE_PAIRS = PAIRS // NW
EXTRA = PAIRS - BASE_PAIRS * NW

OCHUNK = 4096
NCHUNKS = NBT // OCHUNK


def _emb_body(wt_hbm, xt_hbm, out_hbm, row_v, idx_v, out_v, sem):
    wid = lax.axis_index("s") * NC + lax.axis_index("c")
    p0 = wid * BASE_PAIRS + jnp.minimum(wid, EXTRA)
    cnt = BASE_PAIRS + jnp.where(wid < EXTRA, 1, 0)

    def pair_body(k, prev_i):
        p = p0 + k
        i = p // EMBED
        e = p % EMBED

        @pl.when(i != prev_i)
        def _():
            pltpu.sync_copy(xt_hbm.at[i, pl.ds(0, NBT)], idx_v)

        pltpu.sync_copy(wt_hbm.at[i, e, pl.ds(0, CARD)], row_v)

        def chunk_body(c, carry):
            def gat(k16, carry2):
                vidx = idx_v[pl.ds(c * OCHUNK + k16 * 16, 16)]
                out_v[pl.ds(k16 * 16, 16)] = plsc.load_gather(row_v, [vidx])
                return carry2

            lax.fori_loop(0, OCHUNK // 16, gat, 0, unroll=4)
            pltpu.sync_copy(out_v, out_hbm.at[p, pl.ds(c * OCHUNK, OCHUNK)])
            return carry

        lax.fori_loop(0, NCHUNKS, chunk_body, 0)
        return i

    lax.fori_loop(0, cnt, pair_body, jnp.int32(-1))


@jax.jit
def _emb_gather(wt, xt):
    mesh = plsc.VectorSubcoreMesh(core_axis_name="c", subcore_axis_name="s")
    return pl.kernel(
        _emb_body,
        out_type=jax.ShapeDtypeStruct((PAIRS, NBT), jnp.float32),
        mesh=mesh,
        scratch_types=[
            pltpu.VMEM((CARD,), jnp.float32),
            pltpu.VMEM((NBT,), jnp.int32),
            pltpu.VMEM((OCHUNK,), jnp.float32),
            pltpu.SemaphoreType.DMA,
        ],
        compiler_params=pltpu.CompilerParams(needs_layout_passes=False),
    )(wt, xt)


def kernel(x, W):
    wt = W.transpose(0, 2, 1)
    xt = x.reshape(NBT, NUM_FIELDS).astype(jnp.int32).T
    out = _emb_gather(wt, xt)
    return out.T.reshape(B, T, NUM_FIELDS * EMBED, 1)
